# Initial kernel scaffold; baseline (speedup 1.0000x reference)
import functools

import jax
import jax.numpy as jnp
from jax import lax
from jax.experimental import pallas as pl
from jax.experimental.pallas import tpu as pltpu
from jax.experimental.pallas import tpu_sc as plsc


def _mimo_embed_sc(xh, table_flat, H, V, D, R):
    info = plsc.get_sparse_core_info()
    NC, NS, NL = info.num_cores, info.num_subcores, info.num_lanes
    NW = NC * NS
    rpw = R // NW
    C = 400
    assert rpw % C == 0

    mesh = plsc.VectorSubcoreMesh(core_axis_name="c", subcore_axis_name="s")

    @functools.partial(
        pl.kernel,
        out_type=jax.ShapeDtypeStruct((R, D), jnp.float32),
        mesh=mesh,
        scratch_types=[
            pltpu.VMEM((H, C), jnp.int32),
            pltpu.VMEM((C, D), jnp.float32),
            pltpu.SemaphoreType.DMA,
        ],
    )
    def k(x_hbm, tab_hbm, out_hbm, idx_v, acc_v, sem):
        wid = lax.axis_index("s") * NC + lax.axis_index("c")

        def chunk(g, carry):
            base = wid * rpw + g * C
            for h in range(H):
                pltpu.sync_copy(x_hbm.at[h, pl.ds(base, C)], idx_v.at[h])

            def off(i, carry2):
                for h in range(1, H):
                    ih = idx_v.at[h]
                    ih[pl.ds(i * NL, NL)] = ih[pl.ds(i * NL, NL)] + h * V
                return carry2

            lax.fori_loop(0, C // NL, off, 0)

            pltpu.async_copy(tab_hbm.at[idx_v.at[0]], acc_v, sem).wait()
            for h in range(1, H):
                pltpu.async_copy(tab_hbm.at[idx_v.at[h]], acc_v, sem,
                                 add=True).wait()

            pltpu.sync_copy(acc_v, out_hbm.at[pl.ds(base, C)])
            return carry

        lax.fori_loop(0, rpw // C, chunk, 0)

    return k(xh, table_flat)


def kernel(x, tables):
    H, V, D = tables.shape
    B, L = x.shape
    R = (B // H) * L
    xh = x.reshape(H, R)
    table_flat = tables.reshape(H * V, D)
    out = _mimo_embed_sc(xh, table_flat, H, V, D, R)
    return out.reshape(B // H, L, D)


# speedup vs baseline: 1.8253x; 1.8253x over previous
import functools

import jax
import jax.numpy as jnp
from jax import lax
from jax.experimental import pallas as pl
from jax.experimental.pallas import tpu as pltpu
from jax.experimental.pallas import tpu_sc as plsc


def _mimo_embed_sc(xh, table_flat, H, V, D, R):
    info = plsc.get_sparse_core_info()
    NC, NS, NL = info.num_cores, info.num_subcores, info.num_lanes
    NW = NC * NS
    rpw = R // NW
    C = 400
    assert rpw % C == 0

    mesh = plsc.VectorSubcoreMesh(core_axis_name="c", subcore_axis_name="s")

    @functools.partial(
        pl.kernel,
        out_type=jax.ShapeDtypeStruct((R, D), jnp.float32),
        mesh=mesh,
        scratch_types=[
            [pltpu.VMEM((C,), jnp.int32) for _ in range(H)],
            pltpu.VMEM((C, D), jnp.float32),
            pltpu.SemaphoreType.DMA,
        ],
        compiler_params=pltpu.CompilerParams(use_tc_tiling_on_sc=False),
    )
    def k(x_hbm, tab_hbm, out_hbm, idx_v, acc_v, sem):
        wid = lax.axis_index("s") * NC + lax.axis_index("c")

        def chunk(g, carry):
            base = wid * rpw + g * C
            for h in range(H):
                pltpu.sync_copy(x_hbm.at[pl.ds(h * R + base, C)], idx_v[h])

            def off(i, carry2):
                for h in range(1, H):
                    ih = idx_v[h]
                    ih[pl.ds(i * NL, NL)] = ih[pl.ds(i * NL, NL)] + h * V
                return carry2

            lax.fori_loop(0, C // NL, off, 0)

            pltpu.async_copy(tab_hbm.at[idx_v[0]], acc_v, sem).wait()
            for h in range(1, H):
                pltpu.async_copy(tab_hbm.at[idx_v[h]], acc_v, sem,
                                 add=True).wait()

            pltpu.sync_copy(acc_v, out_hbm.at[pl.ds(base, C)])
            return carry

        lax.fori_loop(0, rpw // C, chunk, 0)

    return k(xh, table_flat)


def kernel(x, tables):
    H, V, D = tables.shape
    B, L = x.shape
    R = (B // H) * L
    xh = x.reshape(H * R)
    table_flat = tables.reshape(H * V, D)
    out = _mimo_embed_sc(xh, table_flat, H, V, D, R)
    return out.reshape(B // H, L, D)


# speedup vs baseline: 1.9150x; 1.0492x over previous
import functools

import jax
import jax.numpy as jnp
from jax import lax
from jax.experimental import pallas as pl
from jax.experimental.pallas import tpu as pltpu
from jax.experimental.pallas import tpu_sc as plsc


def _mimo_embed_sc(xh, table_flat, H, V, D, R):
    info = plsc.get_sparse_core_info()
    NC, NS, NL = info.num_cores, info.num_subcores, info.num_lanes
    NW = NC * NS
    rpw = R // NW
    NCHUNK = 4
    C = rpw // NCHUNK
    assert rpw % NCHUNK == 0 and C % 8 == 0

    mesh = plsc.VectorSubcoreMesh(core_axis_name="c", subcore_axis_name="s")

    @functools.partial(
        pl.kernel,
        out_type=jax.ShapeDtypeStruct((R, D), jnp.float32),
        mesh=mesh,
        scratch_types=[
            [pltpu.VMEM((rpw,), jnp.int32) for _ in range(H)],
            [pltpu.VMEM((C, D), jnp.float32) for _ in range(NCHUNK)],
            pltpu.SemaphoreType.DMA,
            [pltpu.SemaphoreType.DMA for _ in range(NCHUNK)],
            [pltpu.SemaphoreType.DMA for _ in range(NCHUNK)],
            [pltpu.SemaphoreType.DMA for _ in range(NCHUNK)],
        ],
        compiler_params=pltpu.CompilerParams(use_tc_tiling_on_sc=False),
    )
    def k(x_hbm, tab_hbm, out_hbm, idx_v, acc_v, sem_i, sem_g0, sem_ga,
          sem_o):
        wid = lax.axis_index("s") * NC + lax.axis_index("c")
        base = wid * rpw

        idx_cp = [
            pltpu.async_copy(x_hbm.at[pl.ds(h * R + base, rpw)], idx_v[h],
                             sem_i)
            for h in range(H)
        ]
        for cp in idx_cp:
            cp.wait()

        def off(h):
            def body(i, carry):
                ih = idx_v[h]
                ih[pl.ds(i * NL, NL)] = ih[pl.ds(i * NL, NL)] + h * V
                return carry
            return body

        for h in range(1, H):
            lax.fori_loop(0, rpw // NL, off(h), 0)

        g0 = [
            pltpu.async_copy(
                tab_hbm.at[idx_v[0].at[pl.ds(g * C, C)]], acc_v[g],
                sem_g0[g])
            for g in range(NCHUNK)
        ]
        ga = []
        for g in range(NCHUNK):
            g0[g].wait()
            ga.append([
                pltpu.async_copy(
                    tab_hbm.at[idx_v[h].at[pl.ds(g * C, C)]], acc_v[g],
                    sem_ga[g], add=True)
                for h in range(1, H)
            ])
        ow = []
        for g in range(NCHUNK):
            for cp in ga[g]:
                cp.wait()
            ow.append(
                pltpu.async_copy(acc_v[g], out_hbm.at[pl.ds(base + g * C, C)],
                                 sem_o[g]))
        for cp in ow:
            cp.wait()

    return k(xh, table_flat)


def kernel(x, tables):
    H, V, D = tables.shape
    B, L = x.shape
    R = (B // H) * L
    xh = x.reshape(H * R)
    table_flat = tables.reshape(H * V, D)
    out = _mimo_embed_sc(xh, table_flat, H, V, D, R)
    return out.reshape(B // H, L, D)
